# baseline (speedup 1.0000x reference)
import jax
import jax.numpy as jnp
import numpy as np
from jax.experimental import pallas as pl
from jax.experimental.pallas import tpu as pltpu

N = 1024
FEAT = 128
HID = 128
HEADS = 8
CH = HID // HEADS


def _expand_proj(a):
    tiled = jnp.concatenate([a] * HEADS, axis=1)
    lane = jax.lax.broadcasted_iota(jnp.int32, (HEADS, HID), 1)
    hrow = jax.lax.broadcasted_iota(jnp.int32, (HEADS, HID), 0)
    return jnp.where(lane // CH == hrow, tiled, 0.0)


def _gat2_kernel(x_ref, adj_ref, W1_ref, as1_ref, ad1_ref, b1_ref,
                 W2_ref, as2_ref, ad2_ref, b2_ref, out_ref):
    adj = adj_ref[...]
    row = jax.lax.broadcasted_iota(jnp.int32, (N, N), 0)
    col = jax.lax.broadcasted_iota(jnp.int32, (N, N), 1)
    maskf = jnp.where(jnp.logical_or(row == col, adj != 0.0), 1.0, 0.0)
    ones_col = jnp.ones((N, 1), dtype=jnp.float32)

    def layer(inp, W_ref, as_ref, ad_ref, b_ref):
        h = jnp.dot(inp, W_ref[...], preferred_element_type=jnp.float32)
        Bs = _expand_proj(as_ref[...])
        Bd = _expand_proj(ad_ref[...])
        al_s = jax.lax.dot_general(h, Bs, (((1,), (1,)), ((), ())),
                                   preferred_element_type=jnp.float32)
        al_d_t = jax.lax.dot_general(Bd, h, (((1,), (1,)), ((), ())),
                                     preferred_element_type=jnp.float32)
        al_d = jax.lax.dot_general(h, Bd, (((1,), (1,)), ((), ())),
                                   preferred_element_type=jnp.float32)
        S = jnp.max(al_s, axis=0, keepdims=True)

        rhs1, rhs2 = [], []
        for hd in range(HEADS):
            du = al_s[:, hd:hd + 1] - S[:, hd:hd + 1]
            h_aug = jnp.concatenate(
                [h[:, hd * CH:(hd + 1) * CH], ones_col], axis=1)
            rhs1.append(jnp.exp(du) * h_aug)
            rhs2.append(jnp.exp(0.2 * du) * h_aug)
        CA = CH + 1
        u2h = jnp.concatenate(rhs2, axis=1)
        shared = jax.lax.dot_general(maskf, u2h, (((0,), (0,)), ((), ())),
                                     preferred_element_type=jnp.float32)

        outs = []
        for hd in range(HEADS):
            s_col = al_s[:, hd:hd + 1]
            d_row = al_d_t[hd:hd + 1, :]
            Sh = S[:, hd:hd + 1]
            zc = Sh + al_d[:, hd:hd + 1]
            mhat = jnp.maximum(zc, 0.2 * zc)
            v1 = jnp.exp(zc - mhat)
            v2 = jnp.exp(0.2 * zc - mhat)
            t = s_col + d_row
            M1 = jnp.where(t >= 0.0, maskf, 0.0)
            rhs = jnp.concatenate([rhs1[hd], rhs2[hd]], axis=1)
            mm = jax.lax.dot_general(M1, rhs, (((0,), (0,)), ((), ())),
                                     preferred_element_type=jnp.float32)
            sh = shared[:, hd * CA:(hd + 1) * CA]
            agg = v1 * mm[:, :CA] + v2 * (sh - mm[:, CA:])
            outs.append(agg[:, :CH] / (agg[:, CH:CA] + 1e-16))
        return jnp.concatenate(outs, axis=1) + b_ref[...]

    h1 = layer(x_ref[...], W1_ref, as1_ref, ad1_ref, b1_ref)
    h1 = jnp.where(h1 > 0.0, h1, jnp.exp(jnp.minimum(h1, 0.0)) - 1.0)
    h2 = layer(h1, W2_ref, as2_ref, ad2_ref, b2_ref)
    out_ref[...] = jnp.where(h2 > 0.0, h2, jnp.exp(jnp.minimum(h2, 0.0)) - 1.0)


@jax.jit
def kernel(x, adj, W1, a_src1, a_dst1, b1, W2, a_src2, a_dst2, b2):
    return pl.pallas_call(
        _gat2_kernel,
        out_shape=jax.ShapeDtypeStruct((N, HID), jnp.float32),
    )(x, adj, W1, a_src1, a_dst1, b1.reshape(1, HID),
      W2, a_src2, a_dst2, b2.reshape(1, HID))


# speedup vs baseline: 1.6604x; 1.6604x over previous
import jax
import jax.numpy as jnp
import numpy as np
from jax.experimental import pallas as pl
from jax.experimental.pallas import tpu as pltpu

N = 1024
FEAT = 128
HID = 128
HEADS = 8
CH = HID // HEADS


def _expand_proj(a):
    tiled = jnp.concatenate([a] * HEADS, axis=1)
    lane = jax.lax.broadcasted_iota(jnp.int32, (HEADS, HID), 1)
    hrow = jax.lax.broadcasted_iota(jnp.int32, (HEADS, HID), 0)
    return jnp.where(lane // CH == hrow, tiled, 0.0)


def _gat2_kernel(x_ref, adj_ref, W1_ref, as1_ref, ad1_ref, b1_ref,
                 W2_ref, as2_ref, ad2_ref, b2_ref, out_ref):
    adj = adj_ref[...]
    row = jax.lax.broadcasted_iota(jnp.int32, (N, N), 0)
    col = jax.lax.broadcasted_iota(jnp.int32, (N, N), 1)
    mask_add = jnp.where(jnp.logical_or(row == col, adj != 0.0), 0.0, -1e4)
    ones_col = jnp.ones((N, 1), dtype=jnp.float32)

    def layer(inp, W_ref, as_ref, ad_ref, b_ref):
        h = jnp.dot(inp, W_ref[...], preferred_element_type=jnp.float32)
        Bs = _expand_proj(as_ref[...])
        Bd = _expand_proj(ad_ref[...])
        al_s = jax.lax.dot_general(h, Bs, (((1,), (1,)), ((), ())),
                                   preferred_element_type=jnp.float32)
        al_d_t = jax.lax.dot_general(Bd, h, (((1,), (1,)), ((), ())),
                                     preferred_element_type=jnp.float32)
        S = jnp.max(al_s, axis=0, keepdims=True)
        outs = []
        for hd in range(HEADS):
            s_col = al_s[:, hd:hd + 1]
            d_row = al_d_t[hd:hd + 1, :]
            Sh = S[:, hd:hd + 1]
            z = Sh + d_row
            mhat = jnp.maximum(z, 0.2 * z)
            w1 = s_col + (d_row - mhat)
            w2 = 0.2 * s_col + (0.2 * d_row - mhat)
            p = jnp.exp(jnp.maximum(w1, w2) + mask_add)
            h_aug = jnp.concatenate(
                [h[:, hd * CH:(hd + 1) * CH], ones_col], axis=1)
            o_aug = jax.lax.dot_general(p, h_aug, (((0,), (0,)), ((), ())),
                                        preferred_element_type=jnp.float32)
            outs.append(o_aug[:, :CH] / (o_aug[:, CH:CH + 1] + 1e-16))
        return jnp.concatenate(outs, axis=1) + b_ref[...]

    h1 = layer(x_ref[...], W1_ref, as1_ref, ad1_ref, b1_ref)
    h1 = jnp.where(h1 > 0.0, h1, jnp.exp(jnp.minimum(h1, 0.0)) - 1.0)
    h2 = layer(h1, W2_ref, as2_ref, ad2_ref, b2_ref)
    out_ref[...] = jnp.where(h2 > 0.0, h2, jnp.exp(jnp.minimum(h2, 0.0)) - 1.0)


@jax.jit
def kernel(x, adj, W1, a_src1, a_dst1, b1, W2, a_src2, a_dst2, b2):
    return pl.pallas_call(
        _gat2_kernel,
        out_shape=jax.ShapeDtypeStruct((N, HID), jnp.float32),
    )(x, adj, W1, a_src1, a_dst1, b1.reshape(1, HID),
      W2, a_src2, a_dst2, b2.reshape(1, HID))
